# baseline (speedup 1.0000x reference)
import functools

import jax
import jax.numpy as jnp
from jax import lax
from jax.experimental import pallas as pl
from jax.experimental.pallas import tpu as pltpu
from jax.experimental.pallas import tpu_sc as plsc

N = 10000
E = 320000
IN = 128
H = 256
OUT = 128
NE = 8
G = 16

NPAD = 10240
RB = 512
NR = NPAD // RB

NSUB = 16
NCORE = 2
CH = 128
CPROC = 161
NCHUNK = 164
EPT = NCHUNK * CH
EREAL = E // NSUB
ROWS_PT = NPAD // NSUB


@functools.lru_cache(maxsize=None)
def _make_seg(ngrp):
    gpc = ngrp // NCORE
    _mesh = plsc.VectorSubcoreMesh(core_axis_name="c", subcore_axis_name="s")

    @functools.partial(
        pl.kernel, mesh=_mesh,
        out_type=jax.ShapeDtypeStruct((ngrp * NPAD, 128), jnp.float32),
        scratch_types=[
            pltpu.VMEM((4, 2, CH), jnp.int32),
            pltpu.VMEM((CH, 128), jnp.float32),
            pltpu.VMEM((CH, 128), jnp.float32),
            pltpu.VMEM_SHARED((NPAD, 128), jnp.float32),
            pltpu.SemaphoreType.DMA,
            pltpu.SemaphoreType.DMA,
            pltpu.SemaphoreType.DMA,
            pltpu.SemaphoreType.DMA,
            pltpu.SemaphoreType.DMA,
        ],
    )
    def seg(feat_hbm, idx_hbm, zeros_hbm, out_hbm,
            idxb, rows_a, rows_b, accum, gsem, ssem0, ssem1, isem0, isem1):
        c = lax.axis_index("c")
        s = lax.axis_index("s")
        rows = (rows_a, rows_b)
        ssem = (ssem0, ssem1)
        isem = (isem0, isem1)

        def wait_g(p):
            pltpu.make_async_copy(feat_hbm.at[idxb.at[0].at[0]], rows[p],
                                  gsem).wait()

        def wait_s(p):
            pltpu.make_async_copy(rows[p], accum.at[idxb.at[0].at[1]],
                                  ssem[p]).wait()

        def wait_i(slot, p):
            pltpu.make_async_copy(idx_hbm.at[0].at[0].at[0], idxb.at[slot],
                                  isem[p]).wait()

        for le in range(gpc):
            g = c * gpc + le
            gi = idx_hbm.at[g].at[s]
            pltpu.sync_copy(zeros_hbm, accum.at[pl.ds(s * ROWS_PT, ROWS_PT)])
            plsc.subcore_barrier()

            pltpu.sync_copy(gi.at[0], idxb.at[0])
            pltpu.async_copy(feat_hbm.at[idxb.at[0].at[0]], rows_a, gsem)
            pltpu.async_copy(gi.at[1], idxb.at[1], isem[1])
            pltpu.async_copy(gi.at[2], idxb.at[2], isem[0])
            wait_g(0)
            pltpu.async_copy(rows_a, accum.at[idxb.at[0].at[1]], ssem[0],
                             add=True)
            wait_i(1, 1)
            pltpu.async_copy(feat_hbm.at[idxb.at[1].at[0]], rows_b, gsem)
            pltpu.async_copy(gi.at[3], idxb.at[3], isem[1])

            def body(jo, carry):
                for b in range(4):
                    i = 1 + jo * 4 + b
                    slot = (b + 1) % 4
                    p = (b + 1) % 2
                    pm1 = b % 2
                    wait_g(p)
                    pltpu.async_copy(rows[p], accum.at[idxb.at[slot].at[1]],
                                     ssem[p], add=True)
                    wait_s(pm1)
                    wait_i((b + 2) % 4, pm1)
                    pltpu.async_copy(
                        feat_hbm.at[idxb.at[(b + 2) % 4].at[0]], rows[pm1],
                        gsem)
                    pltpu.async_copy(gi.at[i + 3], idxb.at[b % 4], isem[pm1])
                return carry

            lax.fori_loop(0, (CPROC - 1) // 4, body, 0)
            wait_g(1)
            wait_s(0)
            wait_i(2, 0)
            wait_i(3, 1)
            plsc.subcore_barrier()
            pltpu.sync_copy(
                accum.at[pl.ds(s * ROWS_PT, ROWS_PT)],
                out_hbm.at[pl.ds((c * gpc + le) * NPAD + s * ROWS_PT, ROWS_PT)])

    return seg


def _enc_body(x_ref, w_ref, b_ref, o_ref):
    o_ref[...] = jax.nn.relu(
        jnp.dot(x_ref[...], w_ref[...], preferred_element_type=jnp.float32)
        + b_ref[...])


def _sizefeat_body(src_ref, bt_ref, o1_ref, o2_ref):
    srcv = src_ref[...]
    bt = bt_ref[...]
    npg = [jnp.sum(jnp.where(bt == g, 1.0, 0.0)) for g in range(G)]
    sf1 = jnp.zeros_like(bt)
    sf2 = jnp.zeros_like(bt)
    hi = 0.0
    for g in range(G):
        lo = hi
        hi = hi + npg[g]
        epg = jnp.sum(jnp.where((srcv >= lo) & (srcv < hi), 1.0, 0.0))
        sf1 = jnp.where(bt == g, jnp.log1p(npg[g]), sf1)
        sf2 = jnp.where(bt == g, jnp.log1p(epg), sf2)
    o1_ref[...] = sf1
    o2_ref[...] = sf2


def _router_body(rin_ref, w1_ref, b1_ref, w2_ref, b2_ref, o_ref):
    pre = jax.nn.relu(
        jnp.dot(rin_ref[...], w1_ref[...], preferred_element_type=jnp.float32)
        + b1_ref[...])
    logits = (jnp.dot(pre, w2_ref[...], preferred_element_type=jnp.float32)
              + b2_ref[...])
    mx = jnp.max(logits, axis=-1, keepdims=True)
    ex = jnp.exp(logits - mx)
    probs = ex / jnp.sum(ex, axis=-1, keepdims=True)
    ids = lax.broadcasted_iota(jnp.int32, (RB, NE), 1)
    i1 = jnp.argmax(probs, axis=-1)[:, None]
    p2 = jnp.where(ids == i1, -jnp.inf, probs)
    i2 = jnp.argmax(p2, axis=-1)[:, None]
    mask = jnp.where((ids == i1) | (ids == i2), 1.0, 0.0)
    masked = probs * mask
    sparse = masked / (jnp.sum(masked, axis=-1, keepdims=True) + 1e-9)
    o_ref[...] = sparse.T


def _l0_body(h_ref, a0_ref, a1_ref, w_ref, b_ref, o_ref):
    w = w_ref[0]
    o_ref[0] = jax.nn.relu(
        jnp.dot(h_ref[...], w[:H], preferred_element_type=jnp.float32)
        + jnp.dot(a0_ref[0], w[H:H + 128], preferred_element_type=jnp.float32)
        + jnp.dot(a1_ref[0], w[H + 128:], preferred_element_type=jnp.float32)
        + b_ref[0])


def _l1_body(y_ref, a_ref, w_ref, b_ref, o_ref):
    w = w_ref[0]
    o_ref[0] = jax.nn.relu(
        jnp.dot(y_ref[0], w[:128], preferred_element_type=jnp.float32)
        + jnp.dot(a_ref[0], w[128:], preferred_element_type=jnp.float32)
        + b_ref[0])


def _l2_body(y_ref, a_ref, w_ref, b_ref, sp_ref, o_ref):
    e = pl.program_id(1)
    w = w_ref[0]
    contrib = (
        jnp.dot(y_ref[0], w[:128], preferred_element_type=jnp.float32)
        + jnp.dot(a_ref[0], w[128:], preferred_element_type=jnp.float32)
        + b_ref[0])
    gated = sp_ref[0, 0, 0, :][:, None] * contrib

    @pl.when(e == 0)
    def _init():
        o_ref[...] = gated

    @pl.when(e > 0)
    def _acc():
        o_ref[...] += gated


def kernel(x, edge_index, batch, W_enc, b_enc, W_r1, b_r1, W_r2, b_r2,
           eW1_0, eW2_0, eb_0, eW1_1, eW2_1, eb_1, eW1_2, eW2_2, eb_2):
    f32 = jnp.float32
    src = edge_index[0]
    dst = edge_index[1]

    src_t = jnp.pad(src.reshape(NSUB, EREAL), ((0, 0), (0, EPT - EREAL)))
    dst_t = jnp.pad(dst.reshape(NSUB, EREAL), ((0, 0), (0, EPT - EREAL)),
                    constant_values=N + 100)
    dst4 = dst_t.reshape(1, NSUB, NCHUNK, 1, CH)
    offs8 = (jnp.arange(NE, dtype=jnp.int32) * NPAD)[:, None, None, None, None]
    srcoff8 = (src_t.reshape(1, NSUB, NCHUNK, 1, CH) + offs8)
    idx8 = jnp.concatenate(
        [srcoff8, jnp.broadcast_to(dst4, srcoff8.shape)], axis=3)
    idx2 = jnp.concatenate(
        [srcoff8[:2], jnp.broadcast_to(dst4, srcoff8[:2].shape)], axis=3)
    zeros_sc = jnp.zeros((ROWS_PT, 128), f32)

    xp = jnp.pad(x, ((0, NPAD - N), (0, 0)))
    batchp = jnp.pad(batch, (0, NPAD - N), constant_values=G + 7)

    h = pl.pallas_call(
        _enc_body,
        grid=(NR,),
        in_specs=[
            pl.BlockSpec((RB, IN), lambda r: (r, 0)),
            pl.BlockSpec((IN, H), lambda r: (0, 0)),
            pl.BlockSpec((1, H), lambda r: (0, 0)),
        ],
        out_specs=pl.BlockSpec((RB, H), lambda r: (r, 0)),
        out_shape=jax.ShapeDtypeStruct((NPAD, H), f32),
    )(xp, W_enc.T, b_enc.reshape(1, H))

    src_f = src.reshape(2500, 128).astype(f32)
    bt_f = batchp.reshape(80, 128).astype(f32)
    sf1, sf2 = pl.pallas_call(
        _sizefeat_body,
        grid=(1,),
        in_specs=[
            pl.BlockSpec((2500, 128), lambda i: (0, 0)),
            pl.BlockSpec((80, 128), lambda i: (0, 0)),
        ],
        out_specs=[pl.BlockSpec((80, 128), lambda i: (0, 0)),
                   pl.BlockSpec((80, 128), lambda i: (0, 0))],
        out_shape=[jax.ShapeDtypeStruct((80, 128), f32),
                   jax.ShapeDtypeStruct((80, 128), f32)],
    )(src_f, bt_f)
    sf_nodes = jnp.stack([sf1.reshape(NPAD), sf2.reshape(NPAD)], axis=1)
    rin = jnp.concatenate([h, sf_nodes], axis=1)

    sparse_t = pl.pallas_call(
        _router_body,
        grid=(NR,),
        in_specs=[
            pl.BlockSpec((RB, H + 2), lambda r: (r, 0)),
            pl.BlockSpec((H + 2, H), lambda r: (0, 0)),
            pl.BlockSpec((1, H), lambda r: (0, 0)),
            pl.BlockSpec((H, NE), lambda r: (0, 0)),
            pl.BlockSpec((1, NE), lambda r: (0, 0)),
        ],
        out_specs=pl.BlockSpec((NE, RB), lambda r: (0, r)),
        out_shape=jax.ShapeDtypeStruct((NE, NPAD), f32),
    )(rin, W_r1.T, b_r1.reshape(1, H), W_r2.T, b_r2.reshape(1, NE))

    hcat = jnp.concatenate([h[:, :128], h[:, 128:]], axis=0)
    agg0 = _make_seg(2)(hcat, idx2, zeros_sc).reshape(2, NPAD, 128)

    W0 = jnp.concatenate([jnp.transpose(eW1_0, (0, 2, 1)),
                          jnp.transpose(eW2_0, (0, 2, 1))], axis=1)
    y0 = pl.pallas_call(
        _l0_body,
        grid=(NR, NE),
        in_specs=[
            pl.BlockSpec((RB, H), lambda r, e: (r, 0)),
            pl.BlockSpec((1, RB, 128), lambda r, e: (0, r, 0)),
            pl.BlockSpec((1, RB, 128), lambda r, e: (1, r, 0)),
            pl.BlockSpec((1, H + 256, 128), lambda r, e: (e, 0, 0)),
            pl.BlockSpec((1, 1, 128), lambda r, e: (e, 0, 0)),
        ],
        out_specs=pl.BlockSpec((1, RB, 128), lambda r, e: (e, r, 0)),
        out_shape=jax.ShapeDtypeStruct((NE, NPAD, 128), f32),
    )(h, agg0, agg0, W0, eb_0.reshape(NE, 1, 128))

    agg1 = _make_seg(8)(y0.reshape(NE * NPAD, 128), idx8,
                 zeros_sc).reshape(NE, NPAD, 128)
    W1 = jnp.concatenate([jnp.transpose(eW1_1, (0, 2, 1)),
                          jnp.transpose(eW2_1, (0, 2, 1))], axis=1)
    y1 = pl.pallas_call(
        _l1_body,
        grid=(NR, NE),
        in_specs=[
            pl.BlockSpec((1, RB, 128), lambda r, e: (e, r, 0)),
            pl.BlockSpec((1, RB, 128), lambda r, e: (e, r, 0)),
            pl.BlockSpec((1, 256, 128), lambda r, e: (e, 0, 0)),
            pl.BlockSpec((1, 1, 128), lambda r, e: (e, 0, 0)),
        ],
        out_specs=pl.BlockSpec((1, RB, 128), lambda r, e: (e, r, 0)),
        out_shape=jax.ShapeDtypeStruct((NE, NPAD, 128), f32),
    )(y0, agg1, W1, eb_1.reshape(NE, 1, 128))

    agg2 = _make_seg(8)(y1.reshape(NE * NPAD, 128), idx8,
                 zeros_sc).reshape(NE, NPAD, 128)
    W2 = jnp.concatenate([jnp.transpose(eW1_2, (0, 2, 1)),
                          jnp.transpose(eW2_2, (0, 2, 1))], axis=1)
    out = pl.pallas_call(
        _l2_body,
        grid=(NR, NE),
        in_specs=[
            pl.BlockSpec((1, RB, 128), lambda r, e: (e, r, 0)),
            pl.BlockSpec((1, RB, 128), lambda r, e: (e, r, 0)),
            pl.BlockSpec((1, 256, 128), lambda r, e: (e, 0, 0)),
            pl.BlockSpec((1, 1, 128), lambda r, e: (e, 0, 0)),
            pl.BlockSpec((1, 1, 1, RB), lambda r, e: (e, r, 0, 0)),
        ],
        out_specs=pl.BlockSpec((RB, 128), lambda r, e: (r, 0)),
        out_shape=jax.ShapeDtypeStruct((NPAD, 128), f32),
    )(y1, agg2, W2, eb_2.reshape(NE, 1, 128), sparse_t.reshape(NE, NR, 1, RB))

    return out[:N]


# speedup vs baseline: 1.4481x; 1.4481x over previous
import functools

import jax
import jax.numpy as jnp
from jax import lax
from jax.experimental import pallas as pl
from jax.experimental.pallas import tpu as pltpu
from jax.experimental.pallas import tpu_sc as plsc

N = 10000
E = 320000
IN = 128
H = 256
OUT = 128
NE = 8
G = 16

NPAD = 10240
RB = 512
NR = NPAD // RB

NSUB = 16
NCORE = 2
CH = 128
NCHUNK = 159
EPT = NCHUNK * CH
EREAL = E // NSUB
ROWS_PT = NPAD // NSUB


@functools.lru_cache(maxsize=None)
def _make_seg(ngrp, cols=128, dtype=jnp.float32):
    gpc = ngrp // NCORE
    _mesh = plsc.VectorSubcoreMesh(core_axis_name="c", subcore_axis_name="s")

    @functools.partial(
        pl.kernel, mesh=_mesh,
        out_type=jax.ShapeDtypeStruct((ngrp * NPAD, cols), dtype),
        scratch_types=[
            pltpu.VMEM((2, 2, CH), jnp.int32),
            pltpu.VMEM((CH, cols), dtype),
            pltpu.VMEM((CH, cols), dtype),
            pltpu.VMEM_SHARED((NPAD, cols), dtype),
            pltpu.SemaphoreType.DMA,
            pltpu.SemaphoreType.DMA,
        ],
    )
    def seg(feat_hbm, idx_hbm, zeros_hbm, out_hbm,
            idxb, rows_a, rows_b, accum, gsem, isem):
        c = lax.axis_index("c")
        s = lax.axis_index("s")
        rows = (rows_a, rows_b)

        for le in range(gpc):
            g = c * gpc + le
            pltpu.sync_copy(zeros_hbm, accum.at[pl.ds(s * ROWS_PT, ROWS_PT)])
            plsc.subcore_barrier()

            pltpu.sync_copy(idx_hbm.at[g].at[s].at[0], idxb.at[0])
            pltpu.async_copy(feat_hbm.at[idxb.at[0].at[0]], rows_a, gsem)
            pltpu.async_copy(idx_hbm.at[g].at[s].at[1], idxb.at[1], isem)

            def body(jo, carry):
                for b in range(2):
                    i = jo * 2 + b
                    pltpu.make_async_copy(
                        feat_hbm.at[idxb.at[b].at[0]], rows[b], gsem).wait()
                    pltpu.make_async_copy(
                        idx_hbm.at[g].at[s].at[i + 1], idxb.at[1 - b],
                        isem).wait()
                    pltpu.async_copy(
                        feat_hbm.at[idxb.at[1 - b].at[0]], rows[1 - b], gsem)
                    pltpu.sync_copy(rows[b], accum.at[idxb.at[b].at[1]],
                                    add=True)
                    pltpu.async_copy(
                        idx_hbm.at[g].at[s].at[(i + 2) % NCHUNK], idxb.at[b],
                        isem)
                return carry

            lax.fori_loop(0, (NCHUNK - 1) // 2, body, 0)
            pltpu.make_async_copy(
                feat_hbm.at[idxb.at[0].at[0]], rows_a, gsem).wait()
            pltpu.make_async_copy(
                idx_hbm.at[g].at[s].at[0], idxb.at[1], isem).wait()
            plsc.subcore_barrier()
            pltpu.sync_copy(
                accum.at[pl.ds(s * ROWS_PT, ROWS_PT)],
                out_hbm.at[pl.ds((c * gpc + le) * NPAD + s * ROWS_PT, ROWS_PT)])

    return seg


def _enc_body(x_ref, w_ref, b_ref, o_ref):
    o_ref[...] = jax.nn.relu(
        jnp.dot(x_ref[...], w_ref[...], preferred_element_type=jnp.float32)
        + b_ref[...])


def _sizefeat_body(src_ref, bt_ref, o1_ref, o2_ref):
    srcv = src_ref[...]
    bt = bt_ref[...]
    npg = [jnp.sum(jnp.where(bt == g, 1.0, 0.0)) for g in range(G)]
    sf1 = jnp.zeros_like(bt)
    sf2 = jnp.zeros_like(bt)
    hi = 0.0
    for g in range(G):
        lo = hi
        hi = hi + npg[g]
        epg = jnp.sum(jnp.where((srcv >= lo) & (srcv < hi), 1.0, 0.0))
        sf1 = jnp.where(bt == g, jnp.log1p(npg[g]), sf1)
        sf2 = jnp.where(bt == g, jnp.log1p(epg), sf2)
    o1_ref[...] = sf1
    o2_ref[...] = sf2


def _router_body(rin_ref, w1_ref, b1_ref, w2_ref, b2_ref, o_ref):
    pre = jax.nn.relu(
        jnp.dot(rin_ref[...], w1_ref[...], preferred_element_type=jnp.float32)
        + b1_ref[...])
    logits = (jnp.dot(pre, w2_ref[...], preferred_element_type=jnp.float32)
              + b2_ref[...])
    mx = jnp.max(logits, axis=-1, keepdims=True)
    ex = jnp.exp(logits - mx)
    probs = ex / jnp.sum(ex, axis=-1, keepdims=True)
    ids = lax.broadcasted_iota(jnp.int32, (RB, NE), 1)
    i1 = jnp.argmax(probs, axis=-1)[:, None]
    p2 = jnp.where(ids == i1, -jnp.inf, probs)
    i2 = jnp.argmax(p2, axis=-1)[:, None]
    mask = jnp.where((ids == i1) | (ids == i2), 1.0, 0.0)
    masked = probs * mask
    sparse = masked / (jnp.sum(masked, axis=-1, keepdims=True) + 1e-9)
    o_ref[...] = sparse.T


def _l0_body(h_ref, a0_ref, a1_ref, w_ref, b_ref, o_ref):
    w = w_ref[0]
    o_ref[0] = jax.nn.relu(
        jnp.dot(h_ref[...], w[:H], preferred_element_type=jnp.float32)
        + jnp.dot(a0_ref[0], w[H:H + 128], preferred_element_type=jnp.float32)
        + jnp.dot(a1_ref[0], w[H + 128:], preferred_element_type=jnp.float32)
        + b_ref[0])


def _l1_body(y_ref, a_ref, w_ref, b_ref, o_ref):
    w = w_ref[0]
    o_ref[0] = jax.nn.relu(
        jnp.dot(y_ref[0], w[:128], preferred_element_type=jnp.float32)
        + jnp.dot(a_ref[0], w[128:], preferred_element_type=jnp.float32)
        + b_ref[0])


def _l2_body(y_ref, a_ref, w_ref, b_ref, sp_ref, o_ref):
    e = pl.program_id(1)
    w = w_ref[0]
    contrib = (
        jnp.dot(y_ref[0], w[:128], preferred_element_type=jnp.float32)
        + jnp.dot(a_ref[0], w[128:], preferred_element_type=jnp.float32)
        + b_ref[0])
    gated = sp_ref[0, 0, 0, :][:, None] * contrib

    @pl.when(e == 0)
    def _init():
        o_ref[...] = gated

    @pl.when(e > 0)
    def _acc():
        o_ref[...] += gated


def kernel(x, edge_index, batch, W_enc, b_enc, W_r1, b_r1, W_r2, b_r2,
           eW1_0, eW2_0, eb_0, eW1_1, eW2_1, eb_1, eW1_2, eW2_2, eb_2):
    f32 = jnp.float32
    src = edge_index[0]
    dst = edge_index[1]

    src_t = jnp.pad(src.reshape(NSUB, EREAL), ((0, 0), (0, EPT - EREAL)))
    dst_t = jnp.pad(dst.reshape(NSUB, EREAL), ((0, 0), (0, EPT - EREAL)),
                    constant_values=N + 100)
    dst4 = dst_t.reshape(1, NSUB, NCHUNK, 1, CH)
    offs8 = (jnp.arange(NE, dtype=jnp.int32) * NPAD)[:, None, None, None, None]
    srcoff8 = (src_t.reshape(1, NSUB, NCHUNK, 1, CH) + offs8)
    idx8 = jnp.concatenate(
        [srcoff8, jnp.broadcast_to(dst4, srcoff8.shape)], axis=3)
    idx2 = jnp.concatenate(
        [srcoff8[:2], jnp.broadcast_to(dst4, srcoff8[:2].shape)], axis=3)
    zeros_sc = jnp.zeros((ROWS_PT, 128), f32)

    xp = jnp.pad(x, ((0, NPAD - N), (0, 0)))
    batchp = jnp.pad(batch, (0, NPAD - N), constant_values=G + 7)

    h = pl.pallas_call(
        _enc_body,
        grid=(NR,),
        in_specs=[
            pl.BlockSpec((RB, IN), lambda r: (r, 0)),
            pl.BlockSpec((IN, H), lambda r: (0, 0)),
            pl.BlockSpec((1, H), lambda r: (0, 0)),
        ],
        out_specs=pl.BlockSpec((RB, H), lambda r: (r, 0)),
        out_shape=jax.ShapeDtypeStruct((NPAD, H), f32),
    )(xp, W_enc.T, b_enc.reshape(1, H))

    src_f = src.reshape(2500, 128).astype(f32)
    bt_f = batchp.reshape(80, 128).astype(f32)
    sf1, sf2 = pl.pallas_call(
        _sizefeat_body,
        grid=(1,),
        in_specs=[
            pl.BlockSpec((2500, 128), lambda i: (0, 0)),
            pl.BlockSpec((80, 128), lambda i: (0, 0)),
        ],
        out_specs=[pl.BlockSpec((80, 128), lambda i: (0, 0)),
                   pl.BlockSpec((80, 128), lambda i: (0, 0))],
        out_shape=[jax.ShapeDtypeStruct((80, 128), f32),
                   jax.ShapeDtypeStruct((80, 128), f32)],
    )(src_f, bt_f)
    sf_nodes = jnp.stack([sf1.reshape(NPAD), sf2.reshape(NPAD)], axis=1)
    rin = jnp.concatenate([h, sf_nodes], axis=1)

    sparse_t = pl.pallas_call(
        _router_body,
        grid=(NR,),
        in_specs=[
            pl.BlockSpec((RB, H + 2), lambda r: (r, 0)),
            pl.BlockSpec((H + 2, H), lambda r: (0, 0)),
            pl.BlockSpec((1, H), lambda r: (0, 0)),
            pl.BlockSpec((H, NE), lambda r: (0, 0)),
            pl.BlockSpec((1, NE), lambda r: (0, 0)),
        ],
        out_specs=pl.BlockSpec((NE, RB), lambda r: (0, r)),
        out_shape=jax.ShapeDtypeStruct((NE, NPAD), f32),
    )(rin, W_r1.T, b_r1.reshape(1, H), W_r2.T, b_r2.reshape(1, NE))

    hcat = jnp.concatenate([h[:, :128], h[:, 128:]], axis=0)
    agg0 = _make_seg(2)(hcat, idx2, zeros_sc).reshape(2, NPAD, 128)

    W0 = jnp.concatenate([jnp.transpose(eW1_0, (0, 2, 1)),
                          jnp.transpose(eW2_0, (0, 2, 1))], axis=1)
    y0 = pl.pallas_call(
        _l0_body,
        grid=(NR, NE),
        in_specs=[
            pl.BlockSpec((RB, H), lambda r, e: (r, 0)),
            pl.BlockSpec((1, RB, 128), lambda r, e: (0, r, 0)),
            pl.BlockSpec((1, RB, 128), lambda r, e: (1, r, 0)),
            pl.BlockSpec((1, H + 256, 128), lambda r, e: (e, 0, 0)),
            pl.BlockSpec((1, 1, 128), lambda r, e: (e, 0, 0)),
        ],
        out_specs=pl.BlockSpec((1, RB, 128), lambda r, e: (e, r, 0)),
        out_shape=jax.ShapeDtypeStruct((NE, NPAD, 128), f32),
    )(h, agg0, agg0, W0, eb_0.reshape(NE, 1, 128))

    agg1 = _make_seg(8)(y0.reshape(NE * NPAD, 128), idx8,
                 zeros_sc).reshape(NE, NPAD, 128)
    W1 = jnp.concatenate([jnp.transpose(eW1_1, (0, 2, 1)),
                          jnp.transpose(eW2_1, (0, 2, 1))], axis=1)
    y1 = pl.pallas_call(
        _l1_body,
        grid=(NR, NE),
        in_specs=[
            pl.BlockSpec((1, RB, 128), lambda r, e: (e, r, 0)),
            pl.BlockSpec((1, RB, 128), lambda r, e: (e, r, 0)),
            pl.BlockSpec((1, 256, 128), lambda r, e: (e, 0, 0)),
            pl.BlockSpec((1, 1, 128), lambda r, e: (e, 0, 0)),
        ],
        out_specs=pl.BlockSpec((1, RB, 128), lambda r, e: (e, r, 0)),
        out_shape=jax.ShapeDtypeStruct((NE, NPAD, 128), f32),
    )(y0, agg1, W1, eb_1.reshape(NE, 1, 128))

    agg2 = _make_seg(8)(y1.reshape(NE * NPAD, 128), idx8,
                 zeros_sc).reshape(NE, NPAD, 128)
    W2 = jnp.concatenate([jnp.transpose(eW1_2, (0, 2, 1)),
                          jnp.transpose(eW2_2, (0, 2, 1))], axis=1)
    out = pl.pallas_call(
        _l2_body,
        grid=(NR, NE),
        in_specs=[
            pl.BlockSpec((1, RB, 128), lambda r, e: (e, r, 0)),
            pl.BlockSpec((1, RB, 128), lambda r, e: (e, r, 0)),
            pl.BlockSpec((1, 256, 128), lambda r, e: (e, 0, 0)),
            pl.BlockSpec((1, 1, 128), lambda r, e: (e, 0, 0)),
            pl.BlockSpec((1, 1, 1, RB), lambda r, e: (e, r, 0, 0)),
        ],
        out_specs=pl.BlockSpec((RB, 128), lambda r, e: (r, 0)),
        out_shape=jax.ShapeDtypeStruct((NPAD, 128), f32),
    )(y1, agg2, W2, eb_2.reshape(NE, 1, 128), sparse_t.reshape(NE, NR, 1, RB))

    return out[:N]


# speedup vs baseline: 1.5426x; 1.0653x over previous
import functools

import jax
import jax.numpy as jnp
from jax import lax
from jax.experimental import pallas as pl
from jax.experimental.pallas import tpu as pltpu
from jax.experimental.pallas import tpu_sc as plsc

N = 10000
E = 320000
IN = 128
H = 256
OUT = 128
NE = 8
G = 16

NPAD = 10240
RB = 512
NR = NPAD // RB

NSUB = 16
NCORE = 2
CH = 128
NCHUNK = 159
EPT = NCHUNK * CH
EREAL = E // NSUB
ROWS_PT = NPAD // NSUB


@functools.lru_cache(maxsize=None)
def _make_seg(ngrp, cols=128, dtype=jnp.float32):
    gpc = ngrp // NCORE
    _mesh = plsc.VectorSubcoreMesh(core_axis_name="c", subcore_axis_name="s")

    @functools.partial(
        pl.kernel, mesh=_mesh,
        out_type=jax.ShapeDtypeStruct((ngrp * NPAD, cols), dtype),
        scratch_types=[
            pltpu.VMEM((2, 2, CH), jnp.int32),
            pltpu.VMEM((CH, cols), dtype),
            pltpu.VMEM((CH, cols), dtype),
            pltpu.VMEM_SHARED((NPAD, cols), dtype),
            pltpu.SemaphoreType.DMA,
            pltpu.SemaphoreType.DMA,
        ],
    )
    def seg(feat_hbm, idx_hbm, zeros_hbm, out_hbm,
            idxb, rows_a, rows_b, accum, gsem, isem):
        c = lax.axis_index("c")
        s = lax.axis_index("s")
        rows = (rows_a, rows_b)

        for le in range(gpc):
            g = c * gpc + le
            pltpu.sync_copy(zeros_hbm, accum.at[pl.ds(s * ROWS_PT, ROWS_PT)])
            plsc.subcore_barrier()

            pltpu.sync_copy(idx_hbm.at[g].at[s].at[0], idxb.at[0])
            pltpu.async_copy(feat_hbm.at[idxb.at[0].at[0]], rows_a, gsem)
            pltpu.async_copy(idx_hbm.at[g].at[s].at[1], idxb.at[1], isem)

            def body(jo, carry):
                for b in range(2):
                    i = jo * 2 + b
                    pltpu.make_async_copy(
                        feat_hbm.at[idxb.at[b].at[0]], rows[b], gsem).wait()
                    pltpu.make_async_copy(
                        idx_hbm.at[g].at[s].at[i + 1], idxb.at[1 - b],
                        isem).wait()
                    pltpu.async_copy(
                        feat_hbm.at[idxb.at[1 - b].at[0]], rows[1 - b], gsem)
                    pltpu.sync_copy(rows[b], accum.at[idxb.at[b].at[1]],
                                    add=True)
                    pltpu.async_copy(
                        idx_hbm.at[g].at[s].at[(i + 2) % NCHUNK], idxb.at[b],
                        isem)
                return carry

            lax.fori_loop(0, (NCHUNK - 1) // 2, body, 0)
            pltpu.make_async_copy(
                feat_hbm.at[idxb.at[0].at[0]], rows_a, gsem).wait()
            pltpu.make_async_copy(
                idx_hbm.at[g].at[s].at[0], idxb.at[1], isem).wait()
            plsc.subcore_barrier()
            pltpu.sync_copy(
                accum.at[pl.ds(s * ROWS_PT, ROWS_PT)],
                out_hbm.at[pl.ds((c * gpc + le) * NPAD + s * ROWS_PT, ROWS_PT)])

    return seg


def _enc_body(x_ref, w_ref, b_ref, o_ref):
    o_ref[...] = jax.nn.relu(
        jnp.dot(x_ref[...], w_ref[...], preferred_element_type=jnp.float32)
        + b_ref[...])


def _sizefeat_body(src_ref, bt_ref, o1_ref, o2_ref):
    srcv = src_ref[...]
    bt = bt_ref[...]
    npg = [jnp.sum(jnp.where(bt == g, 1.0, 0.0)) for g in range(G)]
    sf1 = jnp.zeros_like(bt)
    sf2 = jnp.zeros_like(bt)
    hi = 0.0
    for g in range(G):
        lo = hi
        hi = hi + npg[g]
        epg = jnp.sum(jnp.where((srcv >= lo) & (srcv < hi), 1.0, 0.0))
        sf1 = jnp.where(bt == g, jnp.log1p(npg[g]), sf1)
        sf2 = jnp.where(bt == g, jnp.log1p(epg), sf2)
    o1_ref[...] = sf1
    o2_ref[...] = sf2


def _router_body(rin_ref, w1_ref, b1_ref, w2_ref, b2_ref, o_ref):
    pre = jax.nn.relu(
        jnp.dot(rin_ref[...], w1_ref[...], preferred_element_type=jnp.float32)
        + b1_ref[...])
    logits = (jnp.dot(pre, w2_ref[...], preferred_element_type=jnp.float32)
              + b2_ref[...])
    mx = jnp.max(logits, axis=-1, keepdims=True)
    ex = jnp.exp(logits - mx)
    probs = ex / jnp.sum(ex, axis=-1, keepdims=True)
    ids = lax.broadcasted_iota(jnp.int32, (RB, NE), 1)
    i1 = jnp.argmax(probs, axis=-1)[:, None]
    p2 = jnp.where(ids == i1, -jnp.inf, probs)
    i2 = jnp.argmax(p2, axis=-1)[:, None]
    mask = jnp.where((ids == i1) | (ids == i2), 1.0, 0.0)
    masked = probs * mask
    sparse = masked / (jnp.sum(masked, axis=-1, keepdims=True) + 1e-9)
    o_ref[...] = sparse.T


def _l0_body(h_ref, a0_ref, a1_ref, w_ref, b_ref, o_ref):
    w = w_ref[0]
    o_ref[0] = jax.nn.relu(
        jnp.dot(h_ref[...], w[:H], preferred_element_type=jnp.float32)
        + jnp.dot(a0_ref[0], w[H:H + 128], preferred_element_type=jnp.float32)
        + jnp.dot(a1_ref[0], w[H + 128:], preferred_element_type=jnp.float32)
        + b_ref[0])


def _l1_body(y_ref, a_ref, w_ref, b_ref, o_ref):
    w = w_ref[0]
    o_ref[0] = jax.nn.relu(
        jnp.dot(y_ref[0], w[:128], preferred_element_type=jnp.float32)
        + jnp.dot(a_ref[0], w[128:], preferred_element_type=jnp.float32)
        + b_ref[0])


def _l2_body(y_ref, a_ref, w_ref, b_ref, sp_ref, o_ref):
    e = pl.program_id(1)
    w = w_ref[0]
    contrib = (
        jnp.dot(y_ref[0], w[:128], preferred_element_type=jnp.float32)
        + jnp.dot(a_ref[0], w[128:], preferred_element_type=jnp.float32)
        + b_ref[0])
    gated = sp_ref[0, 0, 0, :][:, None] * contrib

    @pl.when(e == 0)
    def _init():
        o_ref[...] = gated

    @pl.when(e > 0)
    def _acc():
        o_ref[...] += gated


def _l2b_body(y_ref, a_ref, w_ref, b_ref, sp_ref, prev_ref, o_ref):
    e = pl.program_id(1)
    w = w_ref[0]
    contrib = (
        jnp.dot(y_ref[0], w[:128], preferred_element_type=jnp.float32)
        + jnp.dot(a_ref[0], w[128:], preferred_element_type=jnp.float32)
        + b_ref[0])
    gated = sp_ref[0, 0, 0, :][:, None] * contrib

    @pl.when(e == 0)
    def _init():
        o_ref[...] = prev_ref[...] + gated

    @pl.when(e > 0)
    def _acc():
        o_ref[...] += gated


def kernel(x, edge_index, batch, W_enc, b_enc, W_r1, b_r1, W_r2, b_r2,
           eW1_0, eW2_0, eb_0, eW1_1, eW2_1, eb_1, eW1_2, eW2_2, eb_2):
    f32 = jnp.float32
    src = edge_index[0]
    dst = edge_index[1]

    src_t = jnp.pad(src.reshape(NSUB, EREAL), ((0, 0), (0, EPT - EREAL)))
    dst_t = jnp.pad(dst.reshape(NSUB, EREAL), ((0, 0), (0, EPT - EREAL)),
                    constant_values=N + 100)
    dst4 = dst_t.reshape(1, NSUB, NCHUNK, 1, CH)
    offs8 = (jnp.arange(NE, dtype=jnp.int32) * NPAD)[:, None, None, None, None]
    srcoff8 = (src_t.reshape(1, NSUB, NCHUNK, 1, CH) + offs8)
    idx8 = jnp.concatenate(
        [srcoff8, jnp.broadcast_to(dst4, srcoff8.shape)], axis=3)
    idx2 = jnp.concatenate(
        [srcoff8[:2], jnp.broadcast_to(dst4, srcoff8[:2].shape)], axis=3)
    zeros_sc = jnp.zeros((ROWS_PT, 128), f32)

    xp = jnp.pad(x, ((0, NPAD - N), (0, 0)))
    batchp = jnp.pad(batch, (0, NPAD - N), constant_values=G + 7)

    h = pl.pallas_call(
        _enc_body,
        grid=(NR,),
        in_specs=[
            pl.BlockSpec((RB, IN), lambda r: (r, 0)),
            pl.BlockSpec((IN, H), lambda r: (0, 0)),
            pl.BlockSpec((1, H), lambda r: (0, 0)),
        ],
        out_specs=pl.BlockSpec((RB, H), lambda r: (r, 0)),
        out_shape=jax.ShapeDtypeStruct((NPAD, H), f32),
    )(xp, W_enc.T, b_enc.reshape(1, H))

    src_f = src.reshape(2500, 128).astype(f32)
    bt_f = batchp.reshape(80, 128).astype(f32)
    sf1, sf2 = pl.pallas_call(
        _sizefeat_body,
        grid=(1,),
        in_specs=[
            pl.BlockSpec((2500, 128), lambda i: (0, 0)),
            pl.BlockSpec((80, 128), lambda i: (0, 0)),
        ],
        out_specs=[pl.BlockSpec((80, 128), lambda i: (0, 0)),
                   pl.BlockSpec((80, 128), lambda i: (0, 0))],
        out_shape=[jax.ShapeDtypeStruct((80, 128), f32),
                   jax.ShapeDtypeStruct((80, 128), f32)],
    )(src_f, bt_f)
    sf_nodes = jnp.stack([sf1.reshape(NPAD), sf2.reshape(NPAD)], axis=1)
    rin = jnp.concatenate([h, sf_nodes], axis=1)

    sparse_t = pl.pallas_call(
        _router_body,
        grid=(NR,),
        in_specs=[
            pl.BlockSpec((RB, H + 2), lambda r: (r, 0)),
            pl.BlockSpec((H + 2, H), lambda r: (0, 0)),
            pl.BlockSpec((1, H), lambda r: (0, 0)),
            pl.BlockSpec((H, NE), lambda r: (0, 0)),
            pl.BlockSpec((1, NE), lambda r: (0, 0)),
        ],
        out_specs=pl.BlockSpec((NE, RB), lambda r: (0, r)),
        out_shape=jax.ShapeDtypeStruct((NE, NPAD), f32),
    )(rin, W_r1.T, b_r1.reshape(1, H), W_r2.T, b_r2.reshape(1, NE))

    hcat = jnp.concatenate([h[:, :128], h[:, 128:]], axis=0)
    agg0 = _make_seg(2)(hcat, idx2, zeros_sc).reshape(2, NPAD, 128)

    NH = NE // 2
    W0 = jnp.concatenate([jnp.transpose(eW1_0, (0, 2, 1)),
                          jnp.transpose(eW2_0, (0, 2, 1))], axis=1)
    W1 = jnp.concatenate([jnp.transpose(eW1_1, (0, 2, 1)),
                          jnp.transpose(eW2_1, (0, 2, 1))], axis=1)
    W2 = jnp.concatenate([jnp.transpose(eW1_2, (0, 2, 1)),
                          jnp.transpose(eW2_2, (0, 2, 1))], axis=1)
    b0 = eb_0.reshape(NE, 1, 128)
    b1 = eb_1.reshape(NE, 1, 128)
    b2 = eb_2.reshape(NE, 1, 128)
    sp4 = sparse_t.reshape(NE, NR, 1, RB)
    idx4 = idx8[:NH]
    seg4 = _make_seg(NH)

    def l0_half(k):
        return pl.pallas_call(
            _l0_body,
            grid=(NR, NH),
            in_specs=[
                pl.BlockSpec((RB, H), lambda r, e: (r, 0)),
                pl.BlockSpec((1, RB, 128), lambda r, e: (0, r, 0)),
                pl.BlockSpec((1, RB, 128), lambda r, e: (1, r, 0)),
                pl.BlockSpec((1, H + 256, 128), lambda r, e: (e, 0, 0)),
                pl.BlockSpec((1, 1, 128), lambda r, e: (e, 0, 0)),
            ],
            out_specs=pl.BlockSpec((1, RB, 128), lambda r, e: (e, r, 0)),
            out_shape=jax.ShapeDtypeStruct((NH, NPAD, 128), f32),
        )(h, agg0, agg0, W0[k * NH:(k + 1) * NH], b0[k * NH:(k + 1) * NH])

    def l1_half(k, y0h, agg1h):
        return pl.pallas_call(
            _l1_body,
            grid=(NR, NH),
            in_specs=[
                pl.BlockSpec((1, RB, 128), lambda r, e: (e, r, 0)),
                pl.BlockSpec((1, RB, 128), lambda r, e: (e, r, 0)),
                pl.BlockSpec((1, 256, 128), lambda r, e: (e, 0, 0)),
                pl.BlockSpec((1, 1, 128), lambda r, e: (e, 0, 0)),
            ],
            out_specs=pl.BlockSpec((1, RB, 128), lambda r, e: (e, r, 0)),
            out_shape=jax.ShapeDtypeStruct((NH, NPAD, 128), f32),
        )(y0h, agg1h, W1[k * NH:(k + 1) * NH], b1[k * NH:(k + 1) * NH])

    l2_specs = [
        pl.BlockSpec((1, RB, 128), lambda r, e: (e, r, 0)),
        pl.BlockSpec((1, RB, 128), lambda r, e: (e, r, 0)),
        pl.BlockSpec((1, 256, 128), lambda r, e: (e, 0, 0)),
        pl.BlockSpec((1, 1, 128), lambda r, e: (e, 0, 0)),
        pl.BlockSpec((1, 1, 1, RB), lambda r, e: (e, r, 0, 0)),
    ]

    y0a = l0_half(0)
    y0b = l0_half(1)
    agg1a = seg4(y0a.reshape(NH * NPAD, 128), idx4,
                 zeros_sc).reshape(NH, NPAD, 128)
    agg1b = seg4(y0b.reshape(NH * NPAD, 128), idx4,
                 zeros_sc).reshape(NH, NPAD, 128)
    y1a = l1_half(0, y0a, agg1a)
    y1b = l1_half(1, y0b, agg1b)
    agg2a = seg4(y1a.reshape(NH * NPAD, 128), idx4,
                 zeros_sc).reshape(NH, NPAD, 128)
    agg2b = seg4(y1b.reshape(NH * NPAD, 128), idx4,
                 zeros_sc).reshape(NH, NPAD, 128)

    outa = pl.pallas_call(
        _l2_body,
        grid=(NR, NH),
        in_specs=l2_specs,
        out_specs=pl.BlockSpec((RB, 128), lambda r, e: (r, 0)),
        out_shape=jax.ShapeDtypeStruct((NPAD, 128), f32),
    )(y1a, agg2a, W2[:NH], b2[:NH], sp4[:NH])
    out = pl.pallas_call(
        _l2b_body,
        grid=(NR, NH),
        in_specs=l2_specs + [pl.BlockSpec((RB, 128), lambda r, e: (r, 0))],
        out_specs=pl.BlockSpec((RB, 128), lambda r, e: (r, 0)),
        out_shape=jax.ShapeDtypeStruct((NPAD, 128), f32),
    )(y1b, agg2b, W2[NH:], b2[NH:], sp4[NH:], outa)

    return out[:N]
